# baseline (speedup 1.0000x reference)
import functools

import jax
import jax.numpy as jnp
from jax import lax
from jax.experimental import pallas as pl
from jax.experimental.pallas import tpu as pltpu
from jax.experimental.pallas import tpu_sc as plsc

NN = 10000
EE = 320000
FF = 128
CO = 64
NDEPTH = 3
KCHEB = 6
KSG = 5

NSC = 2
NTILE = 16
CHUNK = 128
NPAD = 10240
ROWS_PER_TILE = NPAD // NTILE
EDGES_PER_TILE = 10240
CPT = EDGES_PER_TILE // CHUNK
WCH = 40
NWIN = CPT // WCH
EPAD = NSC * NTILE * EDGES_PER_TILE

RB = 1024
GRID = NPAD // RB


def _build_segsum(nunits):
    mesh = plsc.VectorSubcoreMesh(core_axis_name="c", subcore_axis_name="s")
    out_type = [jax.ShapeDtypeStruct((NSC, NPAD, FF), jnp.float32)
                for _ in range(nunits)]
    scratch_types = [
        pltpu.VMEM_SHARED((NPAD, FF), jnp.float32),
        pltpu.VMEM((WCH, CHUNK), jnp.int32),
        pltpu.VMEM((WCH, CHUNK), jnp.int32),
        pltpu.VMEM((CHUNK, FF), jnp.float32),
        pltpu.VMEM((CHUNK, FF), jnp.float32),
        pltpu.SemaphoreType.DMA,
        pltpu.SemaphoreType.DMA,
        pltpu.SemaphoreType.DMA,
        pltpu.SemaphoreType.DMA,
    ]

    @functools.partial(pl.kernel, mesh=mesh, out_type=out_type,
                       scratch_types=scratch_types)
    def segsum(*refs):
        ins = refs[:3 * nunits + 1]
        outs = refs[3 * nunits + 1:3 * nunits + 1 + nunits]
        (acc, rowv, colv, gb0, gb1,
         sem0, sem1, sem2, sem3) = refs[3 * nunits + 1 + nunits:]
        zeros_hbm = ins[3 * nunits]
        sc = lax.axis_index("c")
        tid = lax.axis_index("s")
        base = tid * ROWS_PER_TILE

        for u in range(nunits):
            src, rowh, colh = ins[3 * u], ins[3 * u + 1], ins[3 * u + 2]
            pltpu.sync_copy(zeros_hbm.at[pl.ds(base, ROWS_PER_TILE)],
                            acc.at[pl.ds(base, ROWS_PER_TILE)])
            plsc.subcore_barrier()

            for w in range(NWIN):
                pltpu.sync_copy(rowh.at[sc, tid, pl.ds(w * WCH, WCH)], rowv)
                pltpu.sync_copy(colh.at[sc, tid, pl.ds(w * WCH, WCH)], colv)
                pltpu.make_async_copy(src.at[rowv.at[0]], gb0, sem0).start()
                pltpu.make_async_copy(src.at[rowv.at[1]], gb1, sem1).start()

                def body(jj, carry):
                    j0 = 2 * jj
                    j1 = j0 + 1
                    pltpu.make_async_copy(src.at[rowv.at[j0]], gb0,
                                          sem0).wait()
                    pltpu.async_copy(gb0, acc.at[colv.at[j0]], sem2,
                                     add=True)
                    pltpu.make_async_copy(src.at[rowv.at[j1]], gb1,
                                          sem1).wait()
                    pltpu.async_copy(gb1, acc.at[colv.at[j1]], sem3,
                                     add=True)

                    @pl.when(jj < WCH // 2 - 1)
                    def _():
                        pltpu.make_async_copy(gb0, acc.at[colv.at[j0]],
                                              sem2).wait()
                        pltpu.make_async_copy(src.at[rowv.at[j0 + 2]], gb0,
                                              sem0).start()
                        pltpu.make_async_copy(gb1, acc.at[colv.at[j1]],
                                              sem3).wait()
                        pltpu.make_async_copy(src.at[rowv.at[j1 + 2]], gb1,
                                              sem1).start()

                    return carry

                lax.fori_loop(0, WCH // 2, body, 0)
                pltpu.make_async_copy(gb0, acc.at[colv.at[WCH - 2]],
                                      sem2).wait()
                pltpu.make_async_copy(gb1, acc.at[colv.at[WCH - 1]],
                                      sem3).wait()
            plsc.subcore_barrier()
            pltpu.sync_copy(acc.at[pl.ds(base, ROWS_PER_TILE)],
                            outs[u].at[sc, pl.ds(base, ROWS_PER_TILE)])
            plsc.subcore_barrier()

    return segsum


_SEGSUM_CACHE = {}


def _segsum(units, zeros):
    n = len(units)
    if n not in _SEGSUM_CACHE:
        _SEGSUM_CACHE[n] = _build_segsum(n)
    args = []
    for (s, r, c) in units:
        args += [s, r, c]
    args.append(zeros)
    out = _SEGSUM_CACHE[n](*args)
    return list(out) if isinstance(out, (tuple, list)) else [out]


def _bs_row(w=FF):
    return pl.BlockSpec((RB, w), lambda i: (i, 0))


def _bs_prow(w=FF):
    return pl.BlockSpec((NSC, RB, w), lambda i: (0, i, 0))


def _bs_full(*shape):
    return pl.BlockSpec(shape, lambda i: (0,) * len(shape))


def _pc(body, in_specs, out_widths, args):
    outs = pl.pallas_call(
        body,
        grid=(GRID,),
        in_specs=in_specs,
        out_specs=[_bs_row(w) for w in out_widths],
        out_shape=[jax.ShapeDtypeStruct((NPAD, w), jnp.float32)
                   for w in out_widths],
    )(*args)
    return outs


def _elu(v):
    return jnp.where(v > 0, v, jnp.exp(v) - 1.0)


def _k_deg(d0, d1, o_da, o_db, o_ci):
    outdeg = d0[0] + d0[1]
    indeg = d1[0] + d1[1]
    o_da[...] = jnp.where(outdeg > 0, lax.rsqrt(outdeg), 0.0)
    o_db[...] = lax.rsqrt(indeg + 1.0)
    o_ci[...] = 1.0 / jnp.maximum(indeg, 1.0)


def _k_prep(h, da, db, o_a, o_b):
    hh = h[...]
    o_a[...] = da[...] * hh
    o_b[...] = db[...] * hh


def _k_step1(yA, yB, yC, h, g, da, db, ci, W0, W1, Wl, bl, Wr,
             o_acc, o_s, o_tx1, o_srcA, o_g):
    tx1 = -da[...] * (yA[0] + yA[1])
    mean = ci[...] * (yC[0] + yC[1])
    hh = h[...]
    o_acc[...] = jnp.dot(hh, W0[...]) + jnp.dot(tx1, W1[...])
    o_s[...] = _elu(jnp.dot(mean, Wl[...]) + bl[...] + jnp.dot(hh, Wr[...]))
    o_tx1[...] = tx1
    o_srcA[...] = da[...] * tx1
    o_g[...] = db[...] * db[...] * ((yB[0] + yB[1]) + g[...])


def _k_mid(yA, yB, tx0, g, accin, da, db, Wk, o_tx2, o_srcA, o_g, o_acc):
    tx2 = -2.0 * da[...] * (yA[0] + yA[1]) - tx0[...]
    o_tx2[...] = tx2
    o_srcA[...] = da[...] * tx2
    o_g[...] = db[...] * db[...] * ((yB[0] + yB[1]) + g[...])
    o_acc[...] = accin[...] + jnp.dot(tx2, Wk[...])


def _k_final(yA, yB, tx0, g, accin, s, da, db, W5, bc, Wsg, bsg,
             o_h, o_srcA, o_g):
    tx5 = -2.0 * da[...] * (yA[0] + yA[1]) - tx0[...]
    a = _elu(accin[...] + jnp.dot(tx5, W5[...]) + bc[...])
    h5 = db[...] * ((yB[0] + yB[1]) + g[...])
    gout = _elu(jnp.dot(h5, Wsg[...]) + bsg[...])
    hn = (a + s[...] + gout) * (1.0 / 3.0)
    o_h[...] = hn
    o_srcA[...] = da[...] * hn
    o_g[...] = db[...] * hn


def _k_mix1(yAh, yAx, yCh, yCx, yBh, yBx, h3, xx, gh, gx, da, db, ci,
            W0h, W0x, W1h, W1x, mcb, Wlh, Wlx, mbl, Wrh, Wrx,
            o_base, o_gh, o_gx):
    daa = da[...]
    tx1h = -daa * (yAh[0] + yAh[1])
    tx1x = -daa * (yAx[0] + yAx[1])
    hh = h3[...]
    xv = xx[...]
    cheb = (jnp.dot(hh, W0h[...]) + jnp.dot(xv, W0x[...]) +
            jnp.dot(tx1h, W1h[...]) + jnp.dot(tx1x, W1x[...]) + mcb[...])
    cii = ci[...]
    meanh = cii * (yCh[0] + yCh[1])
    meanx = cii * (yCx[0] + yCx[1])
    sage = _elu(jnp.dot(meanh, Wlh[...]) + jnp.dot(meanx, Wlx[...]) +
                mbl[...] + jnp.dot(hh, Wrh[...]) + jnp.dot(xv, Wrx[...]))
    o_base[...] = cheb + sage
    dbb = db[...]
    o_gh[...] = dbb * dbb * ((yBh[0] + yBh[1]) + gh[...])
    o_gx[...] = dbb * dbb * ((yBx[0] + yBx[1]) + gx[...])


def _k_mix_mid(yBh, yBx, gh, gx, db, o_gh, o_gx):
    dbb = db[...]
    o_gh[...] = dbb * dbb * ((yBh[0] + yBh[1]) + gh[...])
    o_gx[...] = dbb * dbb * ((yBx[0] + yBx[1]) + gx[...])


def _k_mix_final(yBh, yBx, gh, gx, base, db, Wsh, Wsx, msb, o_out):
    dbb = db[...]
    h5h = dbb * ((yBh[0] + yBh[1]) + gh[...])
    h5x = dbb * ((yBx[0] + yBx[1]) + gx[...])
    sg = _elu(jnp.dot(h5h, Wsh[...]) + jnp.dot(h5x, Wsx[...]) + msb[...])
    o_out[...] = (base[...] + sg) * (1.0 / 3.0)


def kernel(x, edge_index, cheb_W, cheb_b, sage_Wl, sage_bl, sage_Wr,
           sg_W, sg_b, mix_cheb_W, mix_cheb_b, mix_sage_Wl, mix_sage_bl,
           mix_sage_Wr, mix_sg_W, mix_sg_b):
    f32 = jnp.float32
    x = x.astype(f32)
    xp = jnp.pad(x, ((0, NPAD - NN), (0, 0)))

    row = edge_index[0].astype(jnp.int32)
    col = edge_index[1].astype(jnp.int32)
    ndum = EPAD - EE
    dummy = NN + (jnp.arange(ndum, dtype=jnp.int32) % (NPAD - NN))
    rowp = jnp.concatenate([row, dummy]).reshape(NSC, NTILE, CPT, CHUNK)
    colp = jnp.concatenate([col, dummy]).reshape(NSC, NTILE, CPT, CHUNK)

    zeros = jnp.zeros((NPAD, FF), f32)
    ones = jnp.ones((NPAD, FF), f32)

    d0, d1 = _segsum([(ones, rowp, rowp), (ones, colp, colp)], zeros)
    da, db, ci = _pc(_k_deg, [_bs_prow(), _bs_prow()],
                     [FF, FF, FF], [d0, d1])

    b2 = lambda v: v.reshape(1, -1).astype(f32)

    h = xp
    srcA, g = _pc(_k_prep, [_bs_row(), _bs_row(), _bs_row()],
                  [FF, FF], [h, da, db])
    for i in range(NDEPTH):
        W = cheb_W[i].astype(f32)
        yA, yB, yC = _segsum([(srcA, rowp, colp), (g, rowp, colp),
                              (h, rowp, colp)], zeros)
        acc, s, tx1, srcA, g = _pc(
            _k_step1,
            [_bs_prow(), _bs_prow(), _bs_prow(), _bs_row(), _bs_row(),
             _bs_row(), _bs_row(), _bs_row(), _bs_full(FF, FF),
             _bs_full(FF, FF), _bs_full(FF, FF), _bs_full(1, FF),
             _bs_full(FF, FF)],
            [FF, FF, FF, FF, FF],
            [yA, yB, yC, h, g, da, db, ci, W[0], W[1],
             sage_Wl[i].astype(f32), b2(sage_bl[i]),
             sage_Wr[i].astype(f32)])
        tx0 = h
        for k in range(2, KCHEB):
            yA, yB = _segsum([(srcA, rowp, colp), (g, rowp, colp)], zeros)
            if k < KCHEB - 1:
                tx2, srcA, g, acc = _pc(
                    _k_mid,
                    [_bs_prow(), _bs_prow(), _bs_row(), _bs_row(),
                     _bs_row(), _bs_row(), _bs_row(), _bs_full(FF, FF)],
                    [FF, FF, FF, FF],
                    [yA, yB, tx0, g, acc, da, db, W[k]])
                tx0, tx1 = tx1, tx2
            else:
                h, srcA, g = _pc(
                    _k_final,
                    [_bs_prow(), _bs_prow(), _bs_row(), _bs_row(),
                     _bs_row(), _bs_row(), _bs_row(), _bs_row(),
                     _bs_full(FF, FF), _bs_full(1, FF), _bs_full(FF, FF),
                     _bs_full(1, FF)],
                    [FF, FF, FF],
                    [yA, yB, tx0, g, acc, s, da, db, W[KCHEB - 1],
                     b2(cheb_b[i]), sg_W[i].astype(f32), b2(sg_b[i])])

    srcAx, gx = _pc(_k_prep, [_bs_row(), _bs_row(), _bs_row()],
                    [FF, FF], [xp, da, db])
    gh = g
    yAh, yAx, yCh, yCx, yBh, yBx = _segsum(
        [(srcA, rowp, colp), (srcAx, rowp, colp), (h, rowp, colp),
         (xp, rowp, colp), (gh, rowp, colp), (gx, rowp, colp)], zeros)
    mW = mix_cheb_W.astype(f32)
    base, gh, gx = _pc(
        _k_mix1,
        [_bs_prow()] * 6 + [_bs_row()] * 7 +
        [_bs_full(FF, CO)] * 4 + [_bs_full(1, CO)] +
        [_bs_full(FF, CO)] * 2 + [_bs_full(1, CO)] + [_bs_full(FF, CO)] * 2,
        [CO, FF, FF],
        [yAh, yAx, yCh, yCx, yBh, yBx, h, xp, gh, gx, da, db, ci,
         mW[0, :FF], mW[0, FF:], mW[1, :FF], mW[1, FF:], b2(mix_cheb_b),
         mix_sage_Wl[:FF].astype(f32), mix_sage_Wl[FF:].astype(f32),
         b2(mix_sage_bl),
         mix_sage_Wr[:FF].astype(f32), mix_sage_Wr[FF:].astype(f32)])
    for _ in range(KSG - 2):
        yBh, yBx = _segsum([(gh, rowp, colp), (gx, rowp, colp)], zeros)
        gh, gx = _pc(
            _k_mix_mid,
            [_bs_prow(), _bs_prow(), _bs_row(), _bs_row(), _bs_row()],
            [FF, FF], [yBh, yBx, gh, gx, db])
    yBh, yBx = _segsum([(gh, rowp, colp), (gx, rowp, colp)], zeros)
    (out,) = _pc(
        _k_mix_final,
        [_bs_prow(), _bs_prow(), _bs_row(), _bs_row(), _bs_row(CO),
         _bs_row(), _bs_full(FF, CO), _bs_full(FF, CO), _bs_full(1, CO)],
        [CO],
        [yBh, yBx, gh, gx, base, db,
         mix_sg_W[:FF].astype(f32), mix_sg_W[FF:].astype(f32),
         b2(mix_sg_b)])
    return out[:NN]


# speedup vs baseline: 1.3036x; 1.3036x over previous
import functools

import jax
import jax.numpy as jnp
from jax import lax
from jax.experimental import pallas as pl
from jax.experimental.pallas import tpu as pltpu
from jax.experimental.pallas import tpu_sc as plsc

NN = 10000
EE = 320000
FF = 128
CO = 64
NDEPTH = 3
KCHEB = 6
KSG = 5

NSC = 2
NTILE = 16
CHUNK = 128
NPAD = 10240
ROWS_PER_TILE = NPAD // NTILE
EDGES_PER_TILE = 10240
CPT = EDGES_PER_TILE // CHUNK
WCH = 40
NWIN = CPT // WCH
EPAD = NSC * NTILE * EDGES_PER_TILE

RB = 1024
GRID = NPAD // RB


def _build_segsum(nunits):
    mesh = plsc.VectorSubcoreMesh(core_axis_name="c", subcore_axis_name="s")
    out_type = [jax.ShapeDtypeStruct((NSC, NPAD, FF), jnp.float32)
                for _ in range(nunits)]
    scratch_types = [
        pltpu.VMEM_SHARED((NPAD, FF), jnp.float32),
        pltpu.VMEM((WCH, CHUNK), jnp.int32),
        pltpu.VMEM((WCH, CHUNK), jnp.int32),
        pltpu.VMEM((CHUNK, FF), jnp.float32),
        pltpu.VMEM((CHUNK, FF), jnp.float32),
        pltpu.SemaphoreType.DMA,
        pltpu.SemaphoreType.DMA,
    ]

    @functools.partial(pl.kernel, mesh=mesh, out_type=out_type,
                       scratch_types=scratch_types)
    def segsum(*refs):
        ins = refs[:3 * nunits + 1]
        outs = refs[3 * nunits + 1:3 * nunits + 1 + nunits]
        acc, rowv, colv, gb0, gb1, sem0, sem1 = refs[3 * nunits + 1 + nunits:]
        zeros_hbm = ins[3 * nunits]
        sc = lax.axis_index("c")
        tid = lax.axis_index("s")
        base = tid * ROWS_PER_TILE

        for u in range(nunits):
            src, rowh, colh = ins[3 * u], ins[3 * u + 1], ins[3 * u + 2]
            pltpu.sync_copy(zeros_hbm.at[pl.ds(base, ROWS_PER_TILE)],
                            acc.at[pl.ds(base, ROWS_PER_TILE)])
            plsc.subcore_barrier()

            for w in range(NWIN):
                pltpu.sync_copy(rowh.at[sc, tid, pl.ds(w * WCH, WCH)], rowv)
                pltpu.sync_copy(colh.at[sc, tid, pl.ds(w * WCH, WCH)], colv)
                pltpu.make_async_copy(src.at[rowv.at[0]], gb0, sem0).start()
                pltpu.make_async_copy(src.at[rowv.at[1]], gb1, sem1).start()

                def body(jj, carry):
                    j0 = 2 * jj
                    j1 = j0 + 1
                    pltpu.make_async_copy(src.at[rowv.at[j0]], gb0,
                                          sem0).wait()
                    pltpu.sync_copy(gb0, acc.at[colv.at[j0]], add=True)

                    @pl.when(jj < WCH // 2 - 1)
                    def _():
                        pltpu.make_async_copy(src.at[rowv.at[j0 + 2]], gb0,
                                              sem0).start()

                    pltpu.make_async_copy(src.at[rowv.at[j1]], gb1,
                                          sem1).wait()
                    pltpu.sync_copy(gb1, acc.at[colv.at[j1]], add=True)

                    @pl.when(jj < WCH // 2 - 1)
                    def _():
                        pltpu.make_async_copy(src.at[rowv.at[j1 + 2]], gb1,
                                              sem1).start()

                    return carry

                lax.fori_loop(0, WCH // 2, body, 0)
            plsc.subcore_barrier()
            pltpu.sync_copy(acc.at[pl.ds(base, ROWS_PER_TILE)],
                            outs[u].at[sc, pl.ds(base, ROWS_PER_TILE)])
            plsc.subcore_barrier()

    return segsum


_SEGSUM_CACHE = {}


def _segsum(units, zeros):
    n = len(units)
    if n not in _SEGSUM_CACHE:
        _SEGSUM_CACHE[n] = _build_segsum(n)
    args = []
    for (s, r, c) in units:
        args += [s, r, c]
    args.append(zeros)
    out = _SEGSUM_CACHE[n](*args)
    return list(out) if isinstance(out, (tuple, list)) else [out]


def _bs_row(w=FF):
    return pl.BlockSpec((RB, w), lambda i: (i, 0))


def _bs_prow(w=FF):
    return pl.BlockSpec((NSC, RB, w), lambda i: (0, i, 0))


def _bs_full(*shape):
    return pl.BlockSpec(shape, lambda i: (0,) * len(shape))


def _pc(body, in_specs, out_widths, args):
    outs = pl.pallas_call(
        body,
        grid=(GRID,),
        in_specs=in_specs,
        out_specs=[_bs_row(w) for w in out_widths],
        out_shape=[jax.ShapeDtypeStruct((NPAD, w), jnp.float32)
                   for w in out_widths],
    )(*args)
    return outs


def _elu(v):
    return jnp.where(v > 0, v, jnp.exp(v) - 1.0)


def _k_deg(d0, d1, o_da, o_db, o_ci):
    outdeg = d0[0] + d0[1]
    indeg = d1[0] + d1[1]
    o_da[...] = jnp.where(outdeg > 0, lax.rsqrt(outdeg), 0.0)
    o_db[...] = lax.rsqrt(indeg + 1.0)
    o_ci[...] = 1.0 / jnp.maximum(indeg, 1.0)


def _k_prep(h, da, db, o_a, o_b):
    hh = h[...]
    o_a[...] = da[...] * hh
    o_b[...] = db[...] * hh


def _k_step1(yA, yB, yC, h, g, da, db, ci, W0, W1, Wl, bl, Wr,
             o_acc, o_s, o_tx1, o_srcA, o_g):
    tx1 = -da[...] * (yA[0] + yA[1])
    mean = ci[...] * (yC[0] + yC[1])
    hh = h[...]
    o_acc[...] = jnp.dot(hh, W0[...]) + jnp.dot(tx1, W1[...])
    o_s[...] = _elu(jnp.dot(mean, Wl[...]) + bl[...] + jnp.dot(hh, Wr[...]))
    o_tx1[...] = tx1
    o_srcA[...] = da[...] * tx1
    o_g[...] = db[...] * db[...] * ((yB[0] + yB[1]) + g[...])


def _k_mid(yA, yB, tx0, g, accin, da, db, Wk, o_tx2, o_srcA, o_g, o_acc):
    tx2 = -2.0 * da[...] * (yA[0] + yA[1]) - tx0[...]
    o_tx2[...] = tx2
    o_srcA[...] = da[...] * tx2
    o_g[...] = db[...] * db[...] * ((yB[0] + yB[1]) + g[...])
    o_acc[...] = accin[...] + jnp.dot(tx2, Wk[...])


def _k_final(yA, yB, tx0, g, accin, s, da, db, W5, bc, Wsg, bsg,
             o_h, o_srcA, o_g):
    tx5 = -2.0 * da[...] * (yA[0] + yA[1]) - tx0[...]
    a = _elu(accin[...] + jnp.dot(tx5, W5[...]) + bc[...])
    h5 = db[...] * ((yB[0] + yB[1]) + g[...])
    gout = _elu(jnp.dot(h5, Wsg[...]) + bsg[...])
    hn = (a + s[...] + gout) * (1.0 / 3.0)
    o_h[...] = hn
    o_srcA[...] = da[...] * hn
    o_g[...] = db[...] * hn


def _k_mix1(yAh, yAx, yCh, yCx, yBh, yBx, h3, xx, gh, gx, da, db, ci,
            W0h, W0x, W1h, W1x, mcb, Wlh, Wlx, mbl, Wrh, Wrx,
            o_base, o_gh, o_gx):
    daa = da[...]
    tx1h = -daa * (yAh[0] + yAh[1])
    tx1x = -daa * (yAx[0] + yAx[1])
    hh = h3[...]
    xv = xx[...]
    cheb = (jnp.dot(hh, W0h[...]) + jnp.dot(xv, W0x[...]) +
            jnp.dot(tx1h, W1h[...]) + jnp.dot(tx1x, W1x[...]) + mcb[...])
    cii = ci[...]
    meanh = cii * (yCh[0] + yCh[1])
    meanx = cii * (yCx[0] + yCx[1])
    sage = _elu(jnp.dot(meanh, Wlh[...]) + jnp.dot(meanx, Wlx[...]) +
                mbl[...] + jnp.dot(hh, Wrh[...]) + jnp.dot(xv, Wrx[...]))
    o_base[...] = cheb + sage
    dbb = db[...]
    o_gh[...] = dbb * dbb * ((yBh[0] + yBh[1]) + gh[...])
    o_gx[...] = dbb * dbb * ((yBx[0] + yBx[1]) + gx[...])


def _k_mix_mid(yBh, yBx, gh, gx, db, o_gh, o_gx):
    dbb = db[...]
    o_gh[...] = dbb * dbb * ((yBh[0] + yBh[1]) + gh[...])
    o_gx[...] = dbb * dbb * ((yBx[0] + yBx[1]) + gx[...])


def _k_mix_final(yBh, yBx, gh, gx, base, db, Wsh, Wsx, msb, o_out):
    dbb = db[...]
    h5h = dbb * ((yBh[0] + yBh[1]) + gh[...])
    h5x = dbb * ((yBx[0] + yBx[1]) + gx[...])
    sg = _elu(jnp.dot(h5h, Wsh[...]) + jnp.dot(h5x, Wsx[...]) + msb[...])
    o_out[...] = (base[...] + sg) * (1.0 / 3.0)


def kernel(x, edge_index, cheb_W, cheb_b, sage_Wl, sage_bl, sage_Wr,
           sg_W, sg_b, mix_cheb_W, mix_cheb_b, mix_sage_Wl, mix_sage_bl,
           mix_sage_Wr, mix_sg_W, mix_sg_b):
    f32 = jnp.float32
    x = x.astype(f32)
    xp = jnp.pad(x, ((0, NPAD - NN), (0, 0)))

    row = edge_index[0].astype(jnp.int32)
    col = edge_index[1].astype(jnp.int32)
    ndum = EPAD - EE
    dummy = NN + (jnp.arange(ndum, dtype=jnp.int32) % (NPAD - NN))
    rowp = jnp.concatenate([row, dummy]).reshape(NSC, NTILE, CPT, CHUNK)
    colp = jnp.concatenate([col, dummy]).reshape(NSC, NTILE, CPT, CHUNK)

    zeros = jnp.zeros((NPAD, FF), f32)
    ones = jnp.ones((NPAD, FF), f32)

    d0, d1 = _segsum([(ones, rowp, rowp), (ones, colp, colp)], zeros)
    da, db, ci = _pc(_k_deg, [_bs_prow(), _bs_prow()],
                     [FF, FF, FF], [d0, d1])

    b2 = lambda v: v.reshape(1, -1).astype(f32)

    h = xp
    srcA, g = _pc(_k_prep, [_bs_row(), _bs_row(), _bs_row()],
                  [FF, FF], [h, da, db])
    for i in range(NDEPTH):
        W = cheb_W[i].astype(f32)
        yA, yB, yC = _segsum([(srcA, rowp, colp), (g, rowp, colp),
                              (h, rowp, colp)], zeros)
        acc, s, tx1, srcA, g = _pc(
            _k_step1,
            [_bs_prow(), _bs_prow(), _bs_prow(), _bs_row(), _bs_row(),
             _bs_row(), _bs_row(), _bs_row(), _bs_full(FF, FF),
             _bs_full(FF, FF), _bs_full(FF, FF), _bs_full(1, FF),
             _bs_full(FF, FF)],
            [FF, FF, FF, FF, FF],
            [yA, yB, yC, h, g, da, db, ci, W[0], W[1],
             sage_Wl[i].astype(f32), b2(sage_bl[i]),
             sage_Wr[i].astype(f32)])
        tx0 = h
        for k in range(2, KCHEB):
            yA, yB = _segsum([(srcA, rowp, colp), (g, rowp, colp)], zeros)
            if k < KCHEB - 1:
                tx2, srcA, g, acc = _pc(
                    _k_mid,
                    [_bs_prow(), _bs_prow(), _bs_row(), _bs_row(),
                     _bs_row(), _bs_row(), _bs_row(), _bs_full(FF, FF)],
                    [FF, FF, FF, FF],
                    [yA, yB, tx0, g, acc, da, db, W[k]])
                tx0, tx1 = tx1, tx2
            else:
                h, srcA, g = _pc(
                    _k_final,
                    [_bs_prow(), _bs_prow(), _bs_row(), _bs_row(),
                     _bs_row(), _bs_row(), _bs_row(), _bs_row(),
                     _bs_full(FF, FF), _bs_full(1, FF), _bs_full(FF, FF),
                     _bs_full(1, FF)],
                    [FF, FF, FF],
                    [yA, yB, tx0, g, acc, s, da, db, W[KCHEB - 1],
                     b2(cheb_b[i]), sg_W[i].astype(f32), b2(sg_b[i])])

    srcAx, gx = _pc(_k_prep, [_bs_row(), _bs_row(), _bs_row()],
                    [FF, FF], [xp, da, db])
    gh = g
    yAh, yAx, yCh, yCx, yBh, yBx = _segsum(
        [(srcA, rowp, colp), (srcAx, rowp, colp), (h, rowp, colp),
         (xp, rowp, colp), (gh, rowp, colp), (gx, rowp, colp)], zeros)
    mW = mix_cheb_W.astype(f32)
    base, gh, gx = _pc(
        _k_mix1,
        [_bs_prow()] * 6 + [_bs_row()] * 7 +
        [_bs_full(FF, CO)] * 4 + [_bs_full(1, CO)] +
        [_bs_full(FF, CO)] * 2 + [_bs_full(1, CO)] + [_bs_full(FF, CO)] * 2,
        [CO, FF, FF],
        [yAh, yAx, yCh, yCx, yBh, yBx, h, xp, gh, gx, da, db, ci,
         mW[0, :FF], mW[0, FF:], mW[1, :FF], mW[1, FF:], b2(mix_cheb_b),
         mix_sage_Wl[:FF].astype(f32), mix_sage_Wl[FF:].astype(f32),
         b2(mix_sage_bl),
         mix_sage_Wr[:FF].astype(f32), mix_sage_Wr[FF:].astype(f32)])
    for _ in range(KSG - 2):
        yBh, yBx = _segsum([(gh, rowp, colp), (gx, rowp, colp)], zeros)
        gh, gx = _pc(
            _k_mix_mid,
            [_bs_prow(), _bs_prow(), _bs_row(), _bs_row(), _bs_row()],
            [FF, FF], [yBh, yBx, gh, gx, db])
    yBh, yBx = _segsum([(gh, rowp, colp), (gx, rowp, colp)], zeros)
    (out,) = _pc(
        _k_mix_final,
        [_bs_prow(), _bs_prow(), _bs_row(), _bs_row(), _bs_row(CO),
         _bs_row(), _bs_full(FF, CO), _bs_full(FF, CO), _bs_full(1, CO)],
        [CO],
        [yBh, yBx, gh, gx, base, db,
         mix_sg_W[:FF].astype(f32), mix_sg_W[FF:].astype(f32),
         b2(mix_sg_b)])
    return out[:NN]
